# baseline (speedup 1.0000x reference)
import functools

import jax
import jax.numpy as jnp
import numpy as np
from jax import lax
from jax.experimental import pallas as pl
from jax.experimental.pallas import tpu as pltpu
from jax.experimental.pallas import tpu_sc as plsc

_N = 10000
_D = 256
_DE = 16
_K = 20
_B3 = 3072
_NC, _NS = 2, 16
_NW = _NC * _NS

_NBR_TOT = _B3 * _K

_NPW = _NBR_TOT // _NW
_TPW = _B3 // _NW

_CHUNK = 128
_NITER = _NPW // _CHUNK
_STG = _CHUNK * _K


def _sc_gather(node_feat, node_bf, tppr_feat, edge_feat, nodes,
               nbr_node_bm, nbr_edge_bm):
    mesh = plsc.VectorSubcoreMesh(core_axis_name="c", subcore_axis_name="s")

    @functools.partial(
        pl.kernel,
        out_type=(
            jax.ShapeDtypeStruct((_B3, _D), jnp.float32),
            jax.ShapeDtypeStruct((_B3, _D), jnp.float32),
            jax.ShapeDtypeStruct((_NBR_TOT, _D), jnp.bfloat16),
            jax.ShapeDtypeStruct((_NBR_TOT, _DE), jnp.float32),
        ),
        mesh=mesh,
        compiler_params=pltpu.CompilerParams(use_tc_tiling_on_sc=False,
                                             needs_layout_passes=False),
        scratch_types=[
            pltpu.VMEM((_TPW,), jnp.int32),
            pltpu.VMEM((_TPW, _D), jnp.float32),
            pltpu.VMEM((_TPW, _D), jnp.float32),
            pltpu.VMEM((_STG,), jnp.int32),
            pltpu.VMEM((_STG,), jnp.int32),
            [pltpu.VMEM((_CHUNK,), jnp.int32) for _ in range(2)],
            [pltpu.VMEM((_CHUNK, _D), jnp.bfloat16) for _ in range(2)],
            [pltpu.VMEM((_CHUNK,), jnp.int32) for _ in range(2)],
            [pltpu.VMEM((_CHUNK, _DE), jnp.float32) for _ in range(2)],
            [pltpu.SemaphoreType.DMA for _ in range(2)],
            [pltpu.SemaphoreType.DMA for _ in range(2)],
            pltpu.SemaphoreType.DMA,
        ],
    )
    def gather_kernel(node_hbm, nodebf_hbm, tppr_hbm, edge_hbm, nodes_hbm,
                      nbm_hbm, ebm_hbm,
                      batch_out, tppr_out, nbr_out, edges_out,
                      tidx_v, brow_v, trow_v, stg_n, stg_e,
                      nidx_v, nrow_v, eidx_v, erow_v, gsem, ssem, bsem):
        wid = lax.axis_index("s") * _NC + lax.axis_index("c")
        lanes = jax.lax.iota(jnp.int32, 16)
        tbase = wid * _TPW
        pltpu.sync_copy(nodes_hbm.at[pl.ds(tbase, _TPW)], tidx_v)
        bg = pltpu.async_copy(node_hbm.at[tidx_v], brow_v, bsem)
        tg = pltpu.async_copy(tppr_hbm.at[tidx_v], trow_v, bsem)
        nbase = wid * _NPW

        def stage(c, b):
            off = nbase + c * _CHUNK
            k = off // _B3
            b0 = off % _B3
            pltpu.sync_copy(nbm_hbm.at[pl.ds(b0 * _K, _STG)], stg_n)
            pltpu.sync_copy(ebm_hbm.at[pl.ds(b0 * _K, _STG)], stg_e)
            for i in range(_CHUNK // 16):
                pos = (lanes + (16 * i)) * _K + k
                nidx_v[b][pl.ds(16 * i, 16)] = plsc.load_gather(stg_n, [pos])
                eidx_v[b][pl.ds(16 * i, 16)] = plsc.load_gather(stg_e, [pos])
            return (pltpu.async_copy(nodebf_hbm.at[nidx_v[b]], nrow_v[b],
                                     gsem[b]),
                    pltpu.async_copy(edge_hbm.at[eidx_v[b]], erow_v[b],
                                     gsem[b]))

        g = stage(0, 0)
        scat = [None, None]
        for c in range(_NITER):
            b = c % 2
            nxt = (c + 1) % 2
            gn = None
            if c + 1 < _NITER:
                if scat[nxt] is not None:
                    scat[nxt][0].wait()
                    scat[nxt][1].wait()
                gn = stage(c + 1, nxt)
            g[0].wait()
            g[1].wait()
            off = nbase + c * _CHUNK
            scat[b] = (
                pltpu.async_copy(nrow_v[b], nbr_out.at[pl.ds(off, _CHUNK)],
                                 ssem[b]),
                pltpu.async_copy(erow_v[b], edges_out.at[pl.ds(off, _CHUNK)],
                                 ssem[b]),
            )
            g = gn
        for s in scat:
            s[0].wait()
            s[1].wait()
        bg.wait()
        pltpu.sync_copy(brow_v, batch_out.at[pl.ds(tbase, _TPW)])
        tg.wait()
        pltpu.sync_copy(trow_v, tppr_out.at[pl.ds(tbase, _TPW)])

    return gather_kernel(node_feat, node_bf, tppr_feat, edge_feat, nodes,
                         nbr_node_bm, nbr_edge_bm)


def _cast_body(x_ref, o_ref):
    o_ref[...] = x_ref[...].astype(jnp.bfloat16)


def _cast_call(x):
    n = x.shape[0]
    blk = 2000
    return pl.pallas_call(
        _cast_body,
        grid=(n // blk,),
        in_specs=[pl.BlockSpec((blk, _D), lambda i: (i, 0))],
        out_specs=pl.BlockSpec((blk, _D), lambda i: (i, 0)),
        out_shape=jax.ShapeDtypeStruct((n, _D), jnp.bfloat16),
    )(x)


_BB = 256
_NBLK = _B3 // _BB
_DH = _D // 2

_C0 = np.float32(0.9999994)
_C1 = np.float32(-19.73903)
_C2 = np.float32(64.93044)
_C3 = np.float32(-85.29358)
_C4 = np.float32(58.899586)
_C5 = np.float32(-21.259054)
_INV2PI = np.float32(1.0 / (2.0 * np.pi))


def _fast_cos2pi(u):
    u = u - jnp.round(u)
    t = u * u
    return ((((_C5 * t + _C4) * t + _C3) * t + _C2) * t + _C1) * t + _C0


def _embed_body(src_ref, tppr_ref, ts_ref, nbrt_ref, nbr_ref, edg_ref,
                wq_ref, wnbr_ref, wedg_ref, wtim_ref, tw_ref, tb_ref,
                wo1_ref, bo1_ref, wo2_ref, bo2_ref, out_ref):
    src = src_ref[...]
    tb = tb_ref[...]
    tw = tw_ref[...]
    qt = jnp.cos(tb)
    q = src @ wq_ref[0:_D, :] + qt @ wq_ref[_D:2 * _D, :]
    q = q * np.float32(1.0 / np.sqrt(_DH))
    dt = ts_ref[:, 0:1] - nbrt_ref[...]
    bf = jnp.bfloat16
    f32 = jnp.float32
    wnbr_b = wnbr_ref[...].astype(bf)
    wedg_b = wedg_ref[...].astype(bf)
    wtim_b = wtim_ref[...].astype(bf)
    mm = functools.partial(jax.lax.dot, precision=None,
                           preferred_element_type=f32)
    tw2 = tw * _INV2PI
    tb2 = tb * _INV2PI
    l0, l1, vs = [], [], []
    for k in range(_K):
        tf = _fast_cos2pi(dt[:, k:k + 1] * tw2 + tb2)
        kv = (mm(nbr_ref[k].astype(bf), wnbr_b)
              + mm(edg_ref[k].astype(bf), wedg_b)
              + mm(tf.astype(bf), wtim_b))
        kk = kv[:, 0:_D]
        vs.append(kv[:, _D:2 * _D])
        l0.append(jnp.sum(q[:, 0:_DH] * kk[:, 0:_DH], axis=1, keepdims=True))
        l1.append(jnp.sum(q[:, _DH:_D] * kk[:, _DH:_D], axis=1, keepdims=True))
    lg0 = jnp.concatenate(l0, axis=1)
    lg1 = jnp.concatenate(l1, axis=1)
    e0 = jnp.exp(lg0 - jnp.max(lg0, axis=1, keepdims=True))
    e1 = jnp.exp(lg1 - jnp.max(lg1, axis=1, keepdims=True))
    a0 = e0 / jnp.sum(e0, axis=1, keepdims=True)
    a1 = e1 / jnp.sum(e1, axis=1, keepdims=True)
    agg0 = jnp.zeros((_BB, _DH), jnp.float32)
    agg1 = jnp.zeros((_BB, _DH), jnp.float32)
    for k in range(_K):
        agg0 = agg0 + a0[:, k:k + 1] * vs[k][:, 0:_DH]
        agg1 = agg1 + a1[:, k:k + 1] * vs[k][:, _DH:_D]
    agg = jnp.concatenate([agg0, agg1], axis=1)
    h1 = jnp.maximum(
        agg @ wo1_ref[0:_D, :] + src @ wo1_ref[_D:2 * _D, :] + bo1_ref[...],
        0.0)
    emb = h1 @ wo2_ref[...] + bo2_ref[...]
    out_ref[...] = emb * 0.7 + src * 0.21 + tppr_ref[...] * 0.09


def _embed_call(src, tppr, ts2, nbrt, nbr3, edg3, wq, wnbr, wedg, wtim,
                tw_row, tb_row, wo1, bo1r, wo2, bo2r):
    full = lambda i: (0, 0)
    row = lambda i: (i, 0)
    return pl.pallas_call(
        _embed_body,
        grid=(_NBLK,),
        in_specs=[
            pl.BlockSpec((_BB, _D), row),
            pl.BlockSpec((_BB, _D), row),
            pl.BlockSpec((_BB, 8), row),
            pl.BlockSpec((_BB, _K), row),
            pl.BlockSpec((_K, _BB, _D), lambda i: (0, i, 0)),
            pl.BlockSpec((_K, _BB, _DE), lambda i: (0, i, 0)),
            pl.BlockSpec((2 * _D, _D), full),
            pl.BlockSpec((_D, 2 * _D), full),
            pl.BlockSpec((_DE, 2 * _D), full),
            pl.BlockSpec((_D, 2 * _D), full),
            pl.BlockSpec((1, _D), full),
            pl.BlockSpec((1, _D), full),
            pl.BlockSpec((2 * _D, _D), full),
            pl.BlockSpec((1, _D), full),
            pl.BlockSpec((_D, _D), full),
            pl.BlockSpec((1, _D), full),
        ],
        out_specs=pl.BlockSpec((_BB, _D), row),
        out_shape=jax.ShapeDtypeStruct((_B3, _D), jnp.float32),
    )(src, tppr, ts2, nbrt, nbr3, edg3, wq, wnbr, wedg, wtim,
      tw_row, tb_row, wo1, bo1r, wo2, bo2r)


def _score_body(se_ref, de_ref, ne_ref, wa1_ref, ba1_ref, wa2_ref, ba2_ref,
                pos_ref, neg_ref):
    sa = se_ref[...] @ wa1_ref[0:_D, :]
    hp = jnp.maximum(sa + de_ref[...] @ wa1_ref[_D:2 * _D, :] + ba1_ref[...], 0.0)
    hn = jnp.maximum(sa + ne_ref[...] @ wa1_ref[_D:2 * _D, :] + ba1_ref[...], 0.0)
    pos_ref[...] = hp @ wa2_ref[...] + ba2_ref[...]
    neg_ref[...] = hn @ wa2_ref[...] + ba2_ref[...]


def _score_call(src_e, dst_e, neg_e, wa1, ba1r, wa2, ba2r):
    nb = src_e.shape[0]
    return pl.pallas_call(
        _score_body,
        out_shape=(jax.ShapeDtypeStruct((nb, 1), jnp.float32),
                   jax.ShapeDtypeStruct((nb, 1), jnp.float32)),
    )(src_e, dst_e, neg_e, wa1, ba1r, wa2, ba2r)


def kernel(source_nodes, destination_nodes, negative_nodes, edge_times,
           edge_idxs, n_neighbors, nbr_node_idx, nbr_edge_idx, nbr_times,
           node_feat, edge_feat, tppr_feat, time_w, time_b,
           Wq, Wk, Wv, Wo1, bo1, Wo2, bo2, Wa1, ba1, Wa2, ba2):
    nb = source_nodes.shape[0]
    nodes = jnp.concatenate([source_nodes, destination_nodes,
                             negative_nodes]).astype(jnp.int32)
    nbm = nbr_node_idx.astype(jnp.int32).reshape(-1)
    ebm = nbr_edge_idx.astype(jnp.int32).reshape(-1)

    node_bf = _cast_call(node_feat)
    src_rows, tppr_rows, nbr_rows, edge_rows = _sc_gather(
        node_feat, node_bf, tppr_feat, edge_feat, nodes, nbm, ebm)

    nbr3 = nbr_rows.reshape(_K, _B3, _D)
    edg3 = edge_rows.reshape(_K, _B3, _DE)

    ts3 = jnp.concatenate([edge_times, edge_times, edge_times])
    ts2 = jnp.broadcast_to(ts3[:, None], (_B3, 8))

    wnbr = jnp.concatenate([Wk[0:_D], Wv[0:_D]], axis=1)
    wedg = jnp.concatenate([Wk[_D:_D + _DE], Wv[_D:_D + _DE]], axis=1)
    wtim = jnp.concatenate([Wk[_D + _DE:], Wv[_D + _DE:]], axis=1)
    tw_row = time_w.reshape(1, _D)
    tb_row = time_b.reshape(1, _D)

    emb = _embed_call(src_rows, tppr_rows, ts2, nbr_times, nbr3, edg3,
                      Wq, wnbr, wedg, wtim, tw_row, tb_row,
                      Wo1, bo1.reshape(1, _D), Wo2, bo2.reshape(1, _D))

    pos_col, neg_col = _score_call(emb[:nb], emb[nb:2 * nb], emb[2 * nb:],
                                   Wa1, ba1.reshape(1, _D), Wa2,
                                   ba2.reshape(1, 1))
    return jnp.stack([pos_col[:, 0], neg_col[:, 0]])


# speedup vs baseline: 1.2740x; 1.2740x over previous
import functools

import jax
import jax.numpy as jnp
import numpy as np
from jax import lax
from jax.experimental import pallas as pl
from jax.experimental.pallas import tpu as pltpu
from jax.experimental.pallas import tpu_sc as plsc

_N = 10000
_E = 160000
_D = 256
_DE = 16
_K = 20
_B3 = 3072
_NC, _NS = 2, 16
_NW = _NC * _NS

_NBR_TOT = _B3 * _K

_NPW = _NBR_TOT // _NW
_TPW = _B3 // _NW

_CHUNK = 128
_NITER = _NPW // _CHUNK
_STG = _CHUNK * _K
_EG = 8 * _DE


def _sc_gather(node_feat, tppr_feat, edge_grp, nodes, nbr_node_bm,
               nbr_edge_bm):
    mesh = plsc.VectorSubcoreMesh(core_axis_name="c", subcore_axis_name="s")

    @functools.partial(
        pl.kernel,
        out_type=(
            jax.ShapeDtypeStruct((_B3, _D), jnp.float32),
            jax.ShapeDtypeStruct((_B3, _D), jnp.float32),
            jax.ShapeDtypeStruct((_NBR_TOT, _D), jnp.float32),
            jax.ShapeDtypeStruct((_NBR_TOT, _EG), jnp.float32),
        ),
        mesh=mesh,
        compiler_params=pltpu.CompilerParams(needs_layout_passes=False),
        scratch_types=[
            pltpu.VMEM((_TPW,), jnp.int32),
            pltpu.VMEM((_TPW, _D), jnp.float32),
            pltpu.VMEM((_STG,), jnp.int32),
            pltpu.VMEM((_STG,), jnp.int32),
            [pltpu.VMEM((_CHUNK,), jnp.int32) for _ in range(2)],
            [pltpu.VMEM((_CHUNK, _D), jnp.float32) for _ in range(2)],
            [pltpu.VMEM((_CHUNK,), jnp.int32) for _ in range(2)],
            [pltpu.VMEM((_CHUNK, _EG), jnp.float32) for _ in range(2)],
            [pltpu.SemaphoreType.DMA for _ in range(2)],
            [pltpu.SemaphoreType.DMA for _ in range(2)],
            pltpu.SemaphoreType.DMA,
        ],
    )
    def gather_kernel(node_hbm, tppr_hbm, edge_hbm, nodes_hbm,
                      nbm_hbm, ebm_hbm,
                      batch_out, tppr_out, nbr_out, edges_out,
                      tidx_v, brow_v, stg_n, stg_e,
                      nidx_v, nrow_v, eidx_v, erow_v, gsem, ssem, bsem):
        wid = lax.axis_index("s") * _NC + lax.axis_index("c")
        lanes = jax.lax.iota(jnp.int32, 16)
        tbase = wid * _TPW
        pltpu.sync_copy(nodes_hbm.at[pl.ds(tbase, _TPW)], tidx_v)
        bg = pltpu.async_copy(node_hbm.at[tidx_v], brow_v, bsem)
        nbase = wid * _NPW

        def stage(c, b):
            off = nbase + c * _CHUNK
            k = off // _B3
            b0 = off % _B3
            pltpu.sync_copy(nbm_hbm.at[pl.ds(b0 * _K, _STG)], stg_n)
            pltpu.sync_copy(ebm_hbm.at[pl.ds(b0 * _K, _STG)], stg_e)
            for i in range(_CHUNK // 16):
                pos = (lanes + (16 * i)) * _K + k
                nidx_v[b][pl.ds(16 * i, 16)] = plsc.load_gather(stg_n, [pos])
                eidx_v[b][pl.ds(16 * i, 16)] = lax.shift_right_logical(
                    plsc.load_gather(stg_e, [pos]), 3)
            return (pltpu.async_copy(node_hbm.at[nidx_v[b]], nrow_v[b],
                                     gsem[b]),
                    pltpu.async_copy(edge_hbm.at[eidx_v[b]], erow_v[b],
                                     gsem[b]))

        g = stage(0, 0)
        scat = [None, None]
        for c in range(_NITER):
            b = c % 2
            nxt = (c + 1) % 2
            gn = None
            if c + 1 < _NITER:
                if scat[nxt] is not None:
                    scat[nxt][0].wait()
                    scat[nxt][1].wait()
                gn = stage(c + 1, nxt)
            g[0].wait()
            g[1].wait()
            off = nbase + c * _CHUNK
            scat[b] = (
                pltpu.async_copy(nrow_v[b], nbr_out.at[pl.ds(off, _CHUNK)],
                                 ssem[b]),
                pltpu.async_copy(erow_v[b], edges_out.at[pl.ds(off, _CHUNK)],
                                 ssem[b]),
            )
            g = gn
        for s in scat:
            s[0].wait()
            s[1].wait()
        bg.wait()
        pltpu.sync_copy(brow_v, batch_out.at[pl.ds(tbase, _TPW)])
        pltpu.async_copy(tppr_hbm.at[tidx_v], brow_v, bsem).wait()
        pltpu.sync_copy(brow_v, tppr_out.at[pl.ds(tbase, _TPW)])

    return gather_kernel(node_feat, tppr_feat, edge_grp, nodes,
                         nbr_node_bm, nbr_edge_bm)


_BB = 256
_NBLK = _B3 // _BB
_DH = _D // 2

_C0 = np.float32(0.9999994)
_C1 = np.float32(-19.73903)
_C2 = np.float32(64.93044)
_C3 = np.float32(-85.29358)
_C4 = np.float32(58.899586)
_C5 = np.float32(-21.259054)
_INV2PI = np.float32(1.0 / (2.0 * np.pi))


def _fast_cos2pi(u):
    u = u - jnp.round(u)
    t = u * u
    return ((((_C5 * t + _C4) * t + _C3) * t + _C2) * t + _C1) * t + _C0


def _embed_body(src_ref, tppr_ref, ts_ref, nbrt_ref, nbr_ref, edg_ref,
                emod_ref, wq_ref, wnbr_ref, wedg_ref, wtim_ref, tw_ref,
                tb_ref, wo1_ref, bo1_ref, wo2_ref, bo2_ref, out_ref):
    src = src_ref[...]
    tb = tb_ref[...]
    tw = tw_ref[...]
    qt = jnp.cos(tb)
    q = src @ wq_ref[0:_D, :] + qt @ wq_ref[_D:2 * _D, :]
    q = q * np.float32(1.0 / np.sqrt(_DH))
    dt = ts_ref[:, 0:1] - nbrt_ref[...]
    bf = jnp.bfloat16
    f32 = jnp.float32
    wnbr_b = wnbr_ref[...].astype(bf)
    wedg_b = wedg_ref[...].astype(bf)
    wtim_b = wtim_ref[...].astype(bf)
    mm = functools.partial(jax.lax.dot, precision=None,
                           preferred_element_type=f32)
    tw2 = tw * _INV2PI
    tb2 = tb * _INV2PI
    gidx = (jax.lax.broadcasted_iota(jnp.int32, (1, _EG), 1)
            // _DE).astype(f32)
    l0, l1, vs = [], [], []
    for k in range(_K):
        tf = _fast_cos2pi(dt[:, k:k + 1] * tw2 + tb2)
        e_sel = jnp.where(emod_ref[:, k:k + 1] == gidx, edg_ref[k], 0.0)
        kv = (mm(nbr_ref[k].astype(bf), wnbr_b)
              + mm(e_sel.astype(bf), wedg_b)
              + mm(tf.astype(bf), wtim_b))
        kk = kv[:, 0:_D]
        vs.append(kv[:, _D:2 * _D])
        l0.append(jnp.sum(q[:, 0:_DH] * kk[:, 0:_DH], axis=1, keepdims=True))
        l1.append(jnp.sum(q[:, _DH:_D] * kk[:, _DH:_D], axis=1, keepdims=True))
    lg0 = jnp.concatenate(l0, axis=1)
    lg1 = jnp.concatenate(l1, axis=1)
    e0 = jnp.exp(lg0 - jnp.max(lg0, axis=1, keepdims=True))
    e1 = jnp.exp(lg1 - jnp.max(lg1, axis=1, keepdims=True))
    a0 = e0 / jnp.sum(e0, axis=1, keepdims=True)
    a1 = e1 / jnp.sum(e1, axis=1, keepdims=True)
    agg0 = jnp.zeros((_BB, _DH), jnp.float32)
    agg1 = jnp.zeros((_BB, _DH), jnp.float32)
    for k in range(_K):
        agg0 = agg0 + a0[:, k:k + 1] * vs[k][:, 0:_DH]
        agg1 = agg1 + a1[:, k:k + 1] * vs[k][:, _DH:_D]
    agg = jnp.concatenate([agg0, agg1], axis=1)
    h1 = jnp.maximum(
        agg @ wo1_ref[0:_D, :] + src @ wo1_ref[_D:2 * _D, :] + bo1_ref[...],
        0.0)
    emb = h1 @ wo2_ref[...] + bo2_ref[...]
    out_ref[...] = emb * 0.7 + src * 0.21 + tppr_ref[...] * 0.09


def _embed_call(src, tppr, ts2, nbrt, nbr3, edg3, emod, wq, wnbr, wedg8,
                wtim, tw_row, tb_row, wo1, bo1r, wo2, bo2r):
    full = lambda i: (0, 0)
    row = lambda i: (i, 0)
    return pl.pallas_call(
        _embed_body,
        grid=(_NBLK,),
        in_specs=[
            pl.BlockSpec((_BB, _D), row),
            pl.BlockSpec((_BB, _D), row),
            pl.BlockSpec((_BB, 8), row),
            pl.BlockSpec((_BB, _K), row),
            pl.BlockSpec((_K, _BB, _D), lambda i: (0, i, 0)),
            pl.BlockSpec((_K, _BB, _EG), lambda i: (0, i, 0)),
            pl.BlockSpec((_BB, _K), row),
            pl.BlockSpec((2 * _D, _D), full),
            pl.BlockSpec((_D, 2 * _D), full),
            pl.BlockSpec((_EG, 2 * _D), full),
            pl.BlockSpec((_D, 2 * _D), full),
            pl.BlockSpec((1, _D), full),
            pl.BlockSpec((1, _D), full),
            pl.BlockSpec((2 * _D, _D), full),
            pl.BlockSpec((1, _D), full),
            pl.BlockSpec((_D, _D), full),
            pl.BlockSpec((1, _D), full),
        ],
        out_specs=pl.BlockSpec((_BB, _D), row),
        out_shape=jax.ShapeDtypeStruct((_B3, _D), jnp.float32),
    )(src, tppr, ts2, nbrt, nbr3, edg3, emod, wq, wnbr, wedg8, wtim,
      tw_row, tb_row, wo1, bo1r, wo2, bo2r)


def _score_body(se_ref, de_ref, ne_ref, wa1_ref, ba1_ref, wa2_ref, ba2_ref,
                pos_ref, neg_ref):
    sa = se_ref[...] @ wa1_ref[0:_D, :]
    hp = jnp.maximum(sa + de_ref[...] @ wa1_ref[_D:2 * _D, :] + ba1_ref[...], 0.0)
    hn = jnp.maximum(sa + ne_ref[...] @ wa1_ref[_D:2 * _D, :] + ba1_ref[...], 0.0)
    pos_ref[...] = hp @ wa2_ref[...] + ba2_ref[...]
    neg_ref[...] = hn @ wa2_ref[...] + ba2_ref[...]


def _score_call(src_e, dst_e, neg_e, wa1, ba1r, wa2, ba2r):
    nb = src_e.shape[0]
    return pl.pallas_call(
        _score_body,
        out_shape=(jax.ShapeDtypeStruct((nb, 1), jnp.float32),
                   jax.ShapeDtypeStruct((nb, 1), jnp.float32)),
    )(src_e, dst_e, neg_e, wa1, ba1r, wa2, ba2r)


def kernel(source_nodes, destination_nodes, negative_nodes, edge_times,
           edge_idxs, n_neighbors, nbr_node_idx, nbr_edge_idx, nbr_times,
           node_feat, edge_feat, tppr_feat, time_w, time_b,
           Wq, Wk, Wv, Wo1, bo1, Wo2, bo2, Wa1, ba1, Wa2, ba2):
    nb = source_nodes.shape[0]
    nodes = jnp.concatenate([source_nodes, destination_nodes,
                             negative_nodes]).astype(jnp.int32)
    nbm = nbr_node_idx.astype(jnp.int32).reshape(-1)
    ebm = nbr_edge_idx.astype(jnp.int32).reshape(-1)
    emod = (nbr_edge_idx.astype(jnp.int32) % 8).astype(jnp.float32)
    edge_grp = edge_feat.reshape(_E // 8, _EG)

    src_rows, tppr_rows, nbr_rows, edge_rows = _sc_gather(
        node_feat, tppr_feat, edge_grp, nodes, nbm, ebm)

    nbr3 = nbr_rows.reshape(_K, _B3, _D)
    edg3 = edge_rows.reshape(_K, _B3, _EG)

    ts3 = jnp.concatenate([edge_times, edge_times, edge_times])
    ts2 = jnp.broadcast_to(ts3[:, None], (_B3, 8))

    wnbr = jnp.concatenate([Wk[0:_D], Wv[0:_D]], axis=1)
    wedg = jnp.concatenate([Wk[_D:_D + _DE], Wv[_D:_D + _DE]], axis=1)
    wedg8 = jnp.concatenate([wedg] * 8, axis=0)
    wtim = jnp.concatenate([Wk[_D + _DE:], Wv[_D + _DE:]], axis=1)
    tw_row = time_w.reshape(1, _D)
    tb_row = time_b.reshape(1, _D)

    emb = _embed_call(src_rows, tppr_rows, ts2, nbr_times, nbr3, edg3,
                      emod, Wq, wnbr, wedg8, wtim, tw_row, tb_row,
                      Wo1, bo1.reshape(1, _D), Wo2, bo2.reshape(1, _D))

    pos_col, neg_col = _score_call(emb[:nb], emb[nb:2 * nb], emb[2 * nb:],
                                   Wa1, ba1.reshape(1, _D), Wa2,
                                   ba2.reshape(1, 1))
    return jnp.stack([pos_col[:, 0], neg_col[:, 0]])
